# Initial kernel scaffold; baseline (speedup 1.0000x reference)
import jax
import jax.numpy as jnp
from jax.experimental import pallas as pl


def kernel(x, edge_index, batch, W1, b1, g1, be1, W2, b2, g2, be2, W3, b3, Wp, bp):
    raise NotImplementedError("write your pallas kernel here")


# speedup vs baseline: 10.5194x; 10.5194x over previous
import functools
import math

import jax
import jax.numpy as jnp
from jax import lax
from jax.experimental import pallas as pl
from jax.experimental.pallas import tpu as pltpu
from jax.experimental.pallas import tpu_sc as plsc

_NC = 2
_NS = 16
_NT = _NC * _NS
_BLK = 128
_LW = 128
_G = 16

_BN_SCALE = 1.0 / math.sqrt(1.0 + 1e-5)
_MM = dict(preferred_element_type=jnp.float32, precision=lax.Precision.HIGHEST)


def _sc_mesh():
    return plsc.VectorSubcoreMesh(core_axis_name="c", subcore_axis_name="s")


def _zero_fill(zeros_hbm, buf_v, acc_sh, row0, rpt, width):
    pltpu.sync_copy(zeros_hbm, buf_v)
    n_full, rem = divmod(rpt, _BLK)
    for z in range(n_full):
        pltpu.sync_copy(buf_v, acc_sh.at[pl.ds(row0 + z * _BLK, _BLK)])
    if rem:
        pltpu.sync_copy(buf_v.at[pl.ds(0, rem)],
                        acc_sh.at[pl.ds(row0 + n_full * _BLK, rem)])


def _copy_out(acc_sh, buf_v, out_hbm, c, row0, rpt):
    n_full, rem = divmod(rpt, _BLK)
    for z in range(n_full):
        r = row0 + z * _BLK
        pltpu.sync_copy(acc_sh.at[pl.ds(r, _BLK)], buf_v)
        pltpu.sync_copy(buf_v, out_hbm.at[c, pl.ds(r, _BLK)])
    if rem:
        r = row0 + n_full * _BLK
        pltpu.sync_copy(acc_sh.at[pl.ds(r, rem)], buf_v.at[pl.ds(0, rem)])
        pltpu.sync_copy(buf_v.at[pl.ds(0, rem)], out_hbm.at[c, pl.ds(r, rem)])


def _sc_count(dst_r, ones_blk, zeros_blk, np_rows):
    kpt = dst_r.shape[1]
    rpt = np_rows // _NS

    @functools.partial(
        pl.kernel,
        out_type=jax.ShapeDtypeStruct((_NC, np_rows, 16), jnp.float32),
        mesh=_sc_mesh(),
        scratch_types=[
            pltpu.VMEM((kpt, _BLK), jnp.int32),
            pltpu.VMEM((_BLK, 16), jnp.float32),
            pltpu.VMEM((_BLK, 16), jnp.float32),
            pltpu.VMEM_SHARED((np_rows, 16), jnp.float32),
        ],
    )
    def k(dst_hbm, ones_hbm, zeros_hbm, out_hbm, dst_v, ones_v, buf_v, acc_sh):
        c = lax.axis_index("c")
        s = lax.axis_index("s")
        wid = c * _NS + s
        pltpu.sync_copy(dst_hbm.at[wid], dst_v)
        pltpu.sync_copy(ones_hbm, ones_v)
        row0 = s * rpt
        _zero_fill(zeros_hbm, buf_v, acc_sh, row0, rpt, 16)
        plsc.subcore_barrier()

        @pl.loop(0, kpt)
        def _(j):
            pltpu.sync_copy(ones_v, acc_sh.at[dst_v.at[j]], add=True)

        plsc.subcore_barrier()
        _copy_out(acc_sh, buf_v, out_hbm, c, row0, rpt)

    return k(dst_r, ones_blk, zeros_blk)


def _sc_agg(hp, src_r, dst_r, zeros_blk, np_rows):
    kpt = src_r.shape[1]
    rpt = np_rows // _NS

    @functools.partial(
        pl.kernel,
        out_type=jax.ShapeDtypeStruct((_NC, np_rows, _LW), jnp.float32),
        mesh=_sc_mesh(),
        scratch_types=[
            pltpu.VMEM((kpt, _BLK), jnp.int32),
            pltpu.VMEM((kpt, _BLK), jnp.int32),
            pltpu.VMEM((_BLK, _LW), jnp.float32),
            pltpu.VMEM_SHARED((np_rows, _LW), jnp.float32),
        ],
    )
    def k(hp_hbm, src_hbm, dst_hbm, zeros_hbm, out_hbm,
          src_v, dst_v, buf_v, acc_sh):
        c = lax.axis_index("c")
        s = lax.axis_index("s")
        wid = c * _NS + s
        pltpu.sync_copy(src_hbm.at[wid], src_v)
        pltpu.sync_copy(dst_hbm.at[wid], dst_v)
        row0 = s * rpt
        _zero_fill(zeros_hbm, buf_v, acc_sh, row0, rpt, _LW)
        plsc.subcore_barrier()

        @pl.loop(0, kpt)
        def _(j):
            pltpu.sync_copy(hp_hbm.at[src_v.at[j]], buf_v)
            pltpu.sync_copy(buf_v, acc_sh.at[dst_v.at[j]], add=True)

        plsc.subcore_barrier()
        _copy_out(acc_sh, buf_v, out_hbm, c, row0, rpt)

    return k(hp, src_r, dst_r, zeros_blk)


def _tc_matmul(x, w, rb=1000):
    n, din = x.shape
    dout = w.shape[1]

    def body(x_ref, w_ref, o_ref):
        o_ref[...] = jnp.dot(x_ref[...], w_ref[...], **_MM)

    return pl.pallas_call(
        body,
        grid=(n // rb,),
        in_specs=[pl.BlockSpec((rb, din), lambda i: (i, 0)),
                  pl.BlockSpec((din, dout), lambda i: (0, 0))],
        out_specs=pl.BlockSpec((rb, dout), lambda i: (i, 0)),
        out_shape=jax.ShapeDtypeStruct((n, dout), jnp.float32),
    )(x, w)


def _tc_prep(cnt, h, rb=1000):
    n, dh = h.shape

    def body(cnt_ref, h_ref, dis_ref, hp_ref, d2h_ref):
        deg = cnt_ref[0][:, 0:1] + cnt_ref[1][:, 0:1] + 1.0
        dis = lax.rsqrt(deg)
        hp = h_ref[...] * dis
        dis_ref[...] = dis
        hp_ref[...] = jnp.concatenate(
            [hp, jnp.zeros((rb, _LW - dh), jnp.float32)], axis=1)
        d2h_ref[...] = hp * dis

    return pl.pallas_call(
        body,
        grid=(n // rb,),
        in_specs=[pl.BlockSpec((_NC, rb, 16), lambda i: (0, i, 0)),
                  pl.BlockSpec((rb, dh), lambda i: (i, 0))],
        out_specs=[pl.BlockSpec((rb, 1), lambda i: (i, 0)),
                   pl.BlockSpec((rb, _LW), lambda i: (i, 0)),
                   pl.BlockSpec((rb, dh), lambda i: (i, 0))],
        out_shape=[jax.ShapeDtypeStruct((n, 1), jnp.float32),
                   jax.ShapeDtypeStruct((n, _LW), jnp.float32),
                   jax.ShapeDtypeStruct((n, dh), jnp.float32)],
    )(cnt, h)


def _tc_mid(p, d2h, dis, b, g, be, w, rb=1000):
    n, dp = d2h.shape
    dn = w.shape[1]

    def body(p_ref, d2h_ref, dis_ref, b_ref, g_ref, be_ref, w_ref,
             hp_o, d2h_o):
        dis = dis_ref[...]
        agg = p_ref[0][:, :dp] + p_ref[1][:, :dp]
        conv = dis * agg + d2h_ref[...] + b_ref[...]
        xn = jnp.maximum(conv * (g_ref[...] * _BN_SCALE) + be_ref[...], 0.0)
        hp = jnp.dot(xn, w_ref[...], **_MM) * dis
        hp_o[...] = jnp.concatenate(
            [hp, jnp.zeros((rb, _LW - dn), jnp.float32)], axis=1)
        d2h_o[...] = hp * dis

    return pl.pallas_call(
        body,
        grid=(n // rb,),
        in_specs=[pl.BlockSpec((_NC, rb, _LW), lambda i: (0, i, 0)),
                  pl.BlockSpec((rb, dp), lambda i: (i, 0)),
                  pl.BlockSpec((rb, 1), lambda i: (i, 0)),
                  pl.BlockSpec((1, dp), lambda i: (0, 0)),
                  pl.BlockSpec((1, dp), lambda i: (0, 0)),
                  pl.BlockSpec((1, dp), lambda i: (0, 0)),
                  pl.BlockSpec((dp, dn), lambda i: (0, 0))],
        out_specs=[pl.BlockSpec((rb, _LW), lambda i: (i, 0)),
                   pl.BlockSpec((rb, dn), lambda i: (i, 0))],
        out_shape=[jax.ShapeDtypeStruct((n, _LW), jnp.float32),
                   jax.ShapeDtypeStruct((n, dn), jnp.float32)],
    )(p, d2h, dis, b, g, be, w)


def _tc_final(p, d2h, dis, b, bt, wp, bp):
    n, d = d2h.shape
    dproj = wp.shape[1]

    def body(p_ref, d2h_ref, dis_ref, b_ref, bt_ref, wp_ref, bp_ref,
             h_o, grep_o):
        dis = dis_ref[...]
        agg = p_ref[0][:, :d] + p_ref[1][:, :d]
        h = jnp.maximum(dis * agg + d2h_ref[...] + b_ref[...], 0.0)
        h_o[...] = h
        btc = bt_ref[...]
        sums, maxs, cnts = [], [], []
        for g in range(_G):
            m = btc == g
            sums.append(jnp.sum(jnp.where(m, h, 0.0), axis=0, keepdims=True))
            maxs.append(jnp.max(jnp.where(m, h, -jnp.inf), axis=0,
                                keepdims=True))
            cnts.append(jnp.sum(jnp.where(m, 1.0, 0.0), axis=0,
                                keepdims=True)[:, 0:1])
        gsum = jnp.concatenate(sums, axis=0)
        gmax = jnp.concatenate(maxs, axis=0)
        gcnt = jnp.concatenate(cnts, axis=0)
        gmean = gsum / jnp.maximum(gcnt, 1.0)
        grep = jnp.dot(jnp.concatenate([gmean, gmax], axis=1),
                       wp_ref[...], **_MM) + bp_ref[...]
        grep_o[...] = grep

    return pl.pallas_call(
        body,
        grid=(1,),
        in_specs=[pl.BlockSpec((_NC, n, _LW), lambda i: (0, 0, 0)),
                  pl.BlockSpec((n, d), lambda i: (0, 0)),
                  pl.BlockSpec((n, 1), lambda i: (0, 0)),
                  pl.BlockSpec((1, d), lambda i: (0, 0)),
                  pl.BlockSpec((n, 1), lambda i: (0, 0)),
                  pl.BlockSpec((d + d, dproj), lambda i: (0, 0)),
                  pl.BlockSpec((1, dproj), lambda i: (0, 0))],
        out_specs=[pl.BlockSpec((n, d), lambda i: (0, 0)),
                   pl.BlockSpec((_G, dproj), lambda i: (0, 0))],
        out_shape=[jax.ShapeDtypeStruct((n, d), jnp.float32),
                   jax.ShapeDtypeStruct((_G, dproj), jnp.float32)],
    )(p, d2h, dis, b, bt, wp, bp)


def kernel(x, edge_index, batch, W1, b1, g1, be1, W2, b2, g2, be2,
           W3, b3, Wp, bp):
    n = x.shape[0]
    e = edge_index.shape[1]
    dh = W1.shape[1]
    dout = W3.shape[1]

    np_rows = ((n + 1 + 127) // 128) * 128

    kpt = -(-e // (_NT * _BLK))
    pad = _NT * kpt * _BLK - e
    srcp = jnp.concatenate([edge_index[0], jnp.zeros((pad,), jnp.int32)])
    dstp = jnp.concatenate([edge_index[1], jnp.full((pad,), n, jnp.int32)])
    src_r = srcp.reshape(_NT, kpt, _BLK)
    dst_r = dstp.reshape(_NT, kpt, _BLK)

    ones16 = jnp.ones((_BLK, 16), jnp.float32)
    zeros16 = jnp.zeros((_BLK, 16), jnp.float32)
    zeros_lw = jnp.zeros((_BLK, _LW), jnp.float32)

    cnt = _sc_count(dst_r, ones16, zeros16, np_rows)
    h1 = _tc_matmul(x, W1)
    dis, hp1, d2h1 = _tc_prep(cnt[:, :n, :], h1)

    p1 = _sc_agg(hp1, src_r, dst_r, zeros_lw, np_rows)
    hp2, d2h2 = _tc_mid(p1[:, :n], d2h1, dis, b1.reshape(1, dh),
                        g1.reshape(1, dh), be1.reshape(1, dh), W2)

    p2 = _sc_agg(hp2, src_r, dst_r, zeros_lw, np_rows)
    hp3, d2h3 = _tc_mid(p2[:, :n], d2h2, dis, b2.reshape(1, dh),
                        g2.reshape(1, dh), be2.reshape(1, dh), W3)

    p3 = _sc_agg(hp3, src_r, dst_r, zeros_lw, np_rows)
    h, grep = _tc_final(p3[:, :n], d2h3, dis, b3.reshape(1, dout),
                        batch.reshape(n, 1), Wp, bp.reshape(1, dout))
    return h, grep
